# Initial kernel scaffold; baseline (speedup 1.0000x reference)
import functools

import jax
import jax.numpy as jnp
from jax import lax
from jax.experimental import pallas as pl
from jax.experimental.pallas import tpu as pltpu
from jax.experimental.pallas import tpu_sc as plsc

V = 50
D = 18432
BATCH = 128
PLEN = 50
B = BATCH * PLEN
NC = 2
NS = 16
NW = NC * NS
BPW = B // NW
K = 2
NCHUNK = BPW // K

_mesh = plsc.VectorSubcoreMesh(core_axis_name="c", subcore_axis_name="s")


@functools.partial(
    pl.kernel,
    out_type=jax.ShapeDtypeStruct((B, D), jnp.float32),
    mesh=_mesh,
    scratch_types=[
        pltpu.VMEM((NCHUNK, K), jnp.int32),
        pltpu.VMEM((K, D), jnp.float32),
        pltpu.VMEM((K, D), jnp.float32),
        pltpu.SemaphoreType.DMA,
        pltpu.SemaphoreType.DMA,
        pltpu.SemaphoreType.DMA,
        pltpu.SemaphoreType.DMA,
    ],
)
def _gather_kernel(tbl, idxs, out, idx_v, buf0, buf1, g0, g1, p0, p1):
    wid = lax.axis_index("s") * NC + lax.axis_index("c")
    base = wid * BPW
    pltpu.sync_copy(idxs.at[wid], idx_v)

    def gather(c, buf, sem):
        pltpu.async_copy(tbl.at[idx_v.at[c]], buf, sem)

    def put(c, buf, sem):
        pltpu.async_copy(buf, out.at[pl.ds(base + c * K, K)], sem)

    def wait_gather(buf, sem):
        pltpu.make_async_copy(tbl.at[idx_v.at[0]], buf, sem).wait()

    def wait_put(buf, sem):
        pltpu.make_async_copy(buf, out.at[pl.ds(base, K)], sem).wait()

    gather(0, buf0, g0)
    gather(1, buf1, g1)

    def body(i, carry):
        c0 = 2 * i
        wait_gather(buf0, g0)
        put(c0, buf0, p0)
        wait_gather(buf1, g1)
        put(c0 + 1, buf1, p1)
        wait_put(buf0, p0)
        gather(c0 + 2, buf0, g0)
        wait_put(buf1, p1)
        gather(c0 + 3, buf1, g1)
        return carry

    lax.fori_loop(0, NCHUNK // 2 - 1, body, 0)
    c_last = NCHUNK - 2
    wait_gather(buf0, g0)
    put(c_last, buf0, p0)
    wait_gather(buf1, g1)
    put(c_last + 1, buf1, p1)
    wait_put(buf0, p0)
    wait_put(buf1, p1)


def kernel(prefix, embedding_table):
    idx = prefix.reshape(NW, NCHUNK, K).astype(jnp.int32)
    out = _gather_kernel(embedding_table, idx)
    return out.reshape(BATCH, PLEN, D)


# speedup vs baseline: 1.1136x; 1.1136x over previous
import functools

import jax
import jax.numpy as jnp
from jax import lax
from jax.experimental import pallas as pl
from jax.experimental.pallas import tpu as pltpu
from jax.experimental.pallas import tpu_sc as plsc

V = 50
D = 18432
BATCH = 128
PLEN = 50
B = BATCH * PLEN
NC = 2
NS = 16
NW = NC * NS
BPW = B // NW
K = 2
NCHUNK = BPW // K

@functools.cache
def _build_gather_kernel():
    mesh = plsc.VectorSubcoreMesh(
        core_axis_name="c", subcore_axis_name="s", num_cores=NC, num_subcores=NS
    )
    return functools.partial(
        pl.kernel,
        out_type=jax.ShapeDtypeStruct((B, D), jnp.float32),
        mesh=mesh,
        scratch_types=[
            pltpu.VMEM((NCHUNK, K), jnp.int32),
            pltpu.VMEM((K, D), jnp.float32),
            pltpu.VMEM((K, D), jnp.float32),
            pltpu.SemaphoreType.DMA,
            pltpu.SemaphoreType.DMA,
            pltpu.SemaphoreType.DMA,
            pltpu.SemaphoreType.DMA,
        ],
    )(_gather_body)


def _gather_body(tbl, idxs, out, idx_v, buf0, buf1, g0, g1, p0, p1):
    wid = lax.axis_index("s") * NC + lax.axis_index("c")
    base = wid * BPW
    pltpu.sync_copy(idxs.at[wid], idx_v)

    def gather(c, buf, sem):
        pltpu.async_copy(tbl.at[idx_v.at[c]], buf, sem)

    def put(c, buf, sem):
        pltpu.async_copy(buf, out.at[pl.ds(base + c * K, K)], sem)

    def wait_gather(buf, sem):
        pltpu.make_async_copy(tbl.at[idx_v.at[0]], buf, sem).wait()

    def wait_put(buf, sem):
        pltpu.make_async_copy(buf, out.at[pl.ds(base, K)], sem).wait()

    gather(0, buf0, g0)
    gather(1, buf1, g1)

    def body(i, carry):
        c0 = 2 * i
        wait_gather(buf0, g0)
        put(c0, buf0, p0)
        wait_gather(buf1, g1)
        put(c0 + 1, buf1, p1)
        wait_put(buf0, p0)
        gather(c0 + 2, buf0, g0)
        wait_put(buf1, p1)
        gather(c0 + 3, buf1, g1)
        return carry

    lax.fori_loop(0, NCHUNK // 2 - 1, body, 0)
    c_last = NCHUNK - 2
    wait_gather(buf0, g0)
    put(c_last, buf0, p0)
    wait_gather(buf1, g1)
    put(c_last + 1, buf1, p1)
    wait_put(buf0, p0)
    wait_put(buf1, p1)


def kernel(prefix, embedding_table):
    idx = prefix.reshape(NW, NCHUNK, K).astype(jnp.int32)
    out = _build_gather_kernel()(embedding_table, idx)
    return out.reshape(BATCH, PLEN, D)


# speedup vs baseline: 1.1174x; 1.0034x over previous
import functools

import jax
import jax.numpy as jnp
from jax import lax
from jax.experimental import pallas as pl
from jax.experimental.pallas import tpu as pltpu
from jax.experimental.pallas import tpu_sc as plsc

V = 50
D = 18432
BATCH = 128
PLEN = 50
B = BATCH * PLEN
NC = 2
NS = 16
NW = NC * NS
BPW = B // NW
BPAD = 224
GROUP = 16
NGROUP = BPW // GROUP
NREM = BPW - NGROUP * GROUP
NBUF = 2


def _gather_body(tbl, idxs, out, idx_vm, shared, *rest):
    bufs = rest[:NBUF]
    gsem = rest[NBUF:2 * NBUF]
    psem = rest[2 * NBUF:3 * NBUF]

    s = lax.axis_index("s")
    wid = s * NC + lax.axis_index("c")
    base = wid * BPW
    pltpu.sync_copy(idxs.at[wid], idx_vm)

    @pl.when(s == 0)
    def _stage_table():
        pltpu.sync_copy(tbl, shared)

    plsc.subcore_barrier()

    def gather(iv, b):
        pltpu.async_copy(tbl.at[pl.ds(iv, 1)], bufs[b], gsem[b])

    def put(c, b):
        pltpu.async_copy(bufs[b], out.at[pl.ds(base + c, 1)], psem[b])

    def wait_gather(b):
        pltpu.make_async_copy(tbl.at[pl.ds(0, 1)], bufs[b], gsem[b]).wait()

    def wait_put(b):
        pltpu.make_async_copy(bufs[b], out.at[pl.ds(base, 1)], psem[b]).wait()

    vec0 = idx_vm[pl.ds(0, GROUP)]
    for b in range(NBUF):
        gather(vec0[b], b)

    def body(i, carry):
        c0 = i * GROUP
        vec = idx_vm[pl.ds(pl.multiple_of(c0, GROUP), GROUP)]
        vec_next = idx_vm[pl.ds(pl.multiple_of(c0 + GROUP, GROUP), GROUP)]
        for j in range(GROUP):
            b = j % NBUF
            wait_gather(b)
            put(c0 + j, b)
            wait_put(b)
            nj = j + NBUF
            iv = vec[nj] if nj < GROUP else vec_next[nj - GROUP]
            gather(iv, b)
        return carry

    lax.fori_loop(0, NGROUP, body, 0)
    vec_r = idx_vm[pl.ds(NGROUP * GROUP, GROUP)]
    for j in range(NREM):
        c = NGROUP * GROUP + j
        b = j % NBUF
        wait_gather(b)
        put(c, b)
        wait_put(b)
        if j + NBUF < NREM:
            gather(vec_r[j + NBUF], b)


@functools.cache
def _build_gather_kernel():
    mesh = plsc.VectorSubcoreMesh(
        core_axis_name="c", subcore_axis_name="s", num_cores=NC, num_subcores=NS
    )
    return functools.partial(
        pl.kernel,
        out_type=jax.ShapeDtypeStruct((B, D), jnp.float32),
        mesh=mesh,
        scratch_types=[
            pltpu.VMEM((BPAD,), jnp.int32),
            pltpu.VMEM_SHARED((V, D), jnp.float32),
        ]
        + [pltpu.VMEM((1, D), jnp.float32)] * NBUF
        + [pltpu.SemaphoreType.DMA] * (2 * NBUF),
    )(_gather_body)


def kernel(prefix, embedding_table):
    idx = prefix.reshape(NW, BPW).astype(jnp.int32)
    idx = jnp.pad(idx, ((0, 0), (0, BPAD - BPW)))
    out = _build_gather_kernel()(embedding_table, idx)
    return out.reshape(BATCH, PLEN, D)


# speedup vs baseline: 1.8287x; 1.6365x over previous
import functools

import jax
import jax.numpy as jnp
from jax import lax
from jax.experimental import pallas as pl
from jax.experimental.pallas import tpu as pltpu
from jax.experimental.pallas import tpu_sc as plsc

V = 50
D = 18432
BATCH = 128
PLEN = 50
B = BATCH * PLEN
NC = 2
NS = 16
NW = NC * NS
BPW = B // NW
BPAD = 224
GROUP = 16
NGROUP = BPW // GROUP
NREM = BPW - NGROUP * GROUP
NBUF = 2


def _gather_body(tbl, idxs, out, idx_vm, shared, *rest):
    bufs = rest[:NBUF]
    gsem = rest[NBUF:2 * NBUF]
    psem = rest[2 * NBUF:3 * NBUF]

    s = lax.axis_index("s")
    wid = s * NC + lax.axis_index("c")
    base = wid * BPW
    pltpu.sync_copy(idxs.at[wid], idx_vm)

    @pl.when(s == 0)
    def _stage_table():
        pltpu.sync_copy(tbl, shared)

    plsc.subcore_barrier()

    def gather(iv, b):
        pltpu.async_copy(tbl.at[pl.ds(iv, 1)], bufs[b], gsem[b])

    def put(c, b):
        bb = c // PLEN
        pp = c - bb * PLEN
        pltpu.async_copy(bufs[b], out.at[4 * wid + bb, pl.ds(pp, 1)], psem[b])

    def wait_gather(b):
        pltpu.make_async_copy(tbl.at[pl.ds(0, 1)], bufs[b], gsem[b]).wait()

    def wait_put(b):
        pltpu.make_async_copy(bufs[b], out.at[0, pl.ds(0, 1)], psem[b]).wait()

    vec0 = idx_vm[pl.ds(0, GROUP)]
    for b in range(NBUF):
        gather(vec0[b], b)

    def body(i, carry):
        c0 = i * GROUP
        vec = idx_vm[pl.ds(pl.multiple_of(c0, GROUP), GROUP)]
        vec_next = idx_vm[pl.ds(pl.multiple_of(c0 + GROUP, GROUP), GROUP)]
        for j in range(GROUP):
            b = j % NBUF
            wait_gather(b)
            put(c0 + j, b)
            wait_put(b)
            nj = j + NBUF
            iv = vec[nj] if nj < GROUP else vec_next[nj - GROUP]
            gather(iv, b)
        return carry

    lax.fori_loop(0, NGROUP, body, 0)
    vec_r = idx_vm[pl.ds(NGROUP * GROUP, GROUP)]
    for j in range(NREM):
        c = NGROUP * GROUP + j
        b = j % NBUF
        wait_gather(b)
        put(c, b)
        wait_put(b)
        if j + NBUF < NREM:
            gather(vec_r[j + NBUF], b)


@functools.cache
def _build_gather_kernel():
    mesh = plsc.VectorSubcoreMesh(
        core_axis_name="c", subcore_axis_name="s", num_cores=NC, num_subcores=NS
    )
    return functools.partial(
        pl.kernel,
        out_type=jax.ShapeDtypeStruct((BATCH, PLEN, D), jnp.float32),
        mesh=mesh,
        scratch_types=[
            pltpu.VMEM((BPAD,), jnp.int32),
            pltpu.VMEM_SHARED((V, D), jnp.float32),
        ]
        + [pltpu.VMEM((1, D), jnp.float32)] * NBUF
        + [pltpu.SemaphoreType.DMA] * (2 * NBUF),
    )(_gather_body)


def kernel(prefix, embedding_table):
    idx = prefix.reshape(NW, BPW).astype(jnp.int32)
    idx = jnp.pad(idx, ((0, 0), (0, BPAD - BPW)))
    return _build_gather_kernel()(embedding_table, idx)
